# Initial kernel scaffold; baseline (speedup 1.0000x reference)
import jax
import jax.numpy as jnp
from jax.experimental import pallas as pl


def kernel(node_features, edge_index, edge_features, W1, b1, W2, b2):
    raise NotImplementedError("write your pallas kernel here")


# speedup vs baseline: 1.9067x; 1.9067x over previous
import functools

import jax
import jax.numpy as jnp
from jax import lax
from jax.experimental import pallas as pl
from jax.experimental.pallas import tpu as pltpu
from jax.experimental.pallas import tpu_sc as plsc

N_NODES = 10000
N_EDGES = 320000
D_FEAT = 128
D_EDGE = 16
HIDDEN = 256
OUT = 128

NC = 2
NS = 16
NW = NC * NS
EDGES_PER_W = N_EDGES // NW
CHUNK = 80
NCHUNK = EDGES_PER_W // CHUNK


def _sc_gather(nf, src, tgt):
    mesh = plsc.VectorSubcoreMesh(core_axis_name="c", subcore_axis_name="s")

    @functools.partial(
        pl.kernel,
        mesh=mesh,
        out_type=[
            jax.ShapeDtypeStruct((N_EDGES, D_FEAT), jnp.float32),
            jax.ShapeDtypeStruct((N_EDGES, D_FEAT), jnp.float32),
        ],
        scratch_types=[
            pltpu.VMEM((CHUNK,), jnp.int32),
            pltpu.VMEM((CHUNK, D_FEAT), jnp.float32),
            pltpu.VMEM((CHUNK,), jnp.int32),
            pltpu.VMEM((CHUNK, D_FEAT), jnp.float32),
            pltpu.SemaphoreType.DMA,
            pltpu.SemaphoreType.DMA,
        ],
    )
    def gather_kernel(nf_hbm, src_hbm, tgt_hbm, srcg_hbm, tgtg_hbm,
                      idx_s, rows_s, idx_t, rows_t, sem_s, sem_t):
        wid = lax.axis_index("s") * NC + lax.axis_index("c")
        wbase = wid * EDGES_PER_W

        def body(i, carry):
            base = wbase + i * CHUNK
            pltpu.sync_copy(src_hbm.at[pl.ds(base, CHUNK)], idx_s)
            pltpu.sync_copy(tgt_hbm.at[pl.ds(base, CHUNK)], idx_t)
            cps = pltpu.async_copy(nf_hbm.at[idx_s], rows_s, sem_s)
            cpt = pltpu.async_copy(nf_hbm.at[idx_t], rows_t, sem_t)
            cps.wait()
            cpt.wait()
            pltpu.sync_copy(rows_s, srcg_hbm.at[pl.ds(base, CHUNK)])
            pltpu.sync_copy(rows_t, tgtg_hbm.at[pl.ds(base, CHUNK)])
            return carry

        lax.fori_loop(0, NCHUNK, body, 0)

    return gather_kernel(nf, src, tgt)


BE = 640


def _tc_mlp(srcg, tgtg, ef, w1a, w1b, w1c, b1, w2, b2):
    grid = N_EDGES // BE

    def body(sg_ref, tg_ref, ef_ref, w1a_ref, w1b_ref, w1c_ref, b1_ref,
             w2_ref, b2_ref, o_ref):
        h = jnp.dot(sg_ref[...], w1a_ref[...], preferred_element_type=jnp.float32)
        h = h + jnp.dot(tg_ref[...], w1b_ref[...], preferred_element_type=jnp.float32)
        h = h + jnp.dot(ef_ref[...], w1c_ref[...], preferred_element_type=jnp.float32)
        h = jnp.maximum(h + b1_ref[...], 0.0)
        o_ref[...] = jnp.dot(h, w2_ref[...], preferred_element_type=jnp.float32) + b2_ref[...]

    return pl.pallas_call(
        body,
        grid=(grid,),
        in_specs=[
            pl.BlockSpec((BE, D_FEAT), lambda i: (i, 0)),
            pl.BlockSpec((BE, D_FEAT), lambda i: (i, 0)),
            pl.BlockSpec((BE, D_EDGE), lambda i: (i, 0)),
            pl.BlockSpec((D_FEAT, HIDDEN), lambda i: (0, 0)),
            pl.BlockSpec((D_FEAT, HIDDEN), lambda i: (0, 0)),
            pl.BlockSpec((D_EDGE, HIDDEN), lambda i: (0, 0)),
            pl.BlockSpec((1, HIDDEN), lambda i: (0, 0)),
            pl.BlockSpec((HIDDEN, OUT), lambda i: (0, 0)),
            pl.BlockSpec((1, OUT), lambda i: (0, 0)),
        ],
        out_specs=pl.BlockSpec((BE, OUT), lambda i: (i, 0)),
        out_shape=jax.ShapeDtypeStruct((N_EDGES, OUT), jnp.float32),
    )(srcg, tgtg, ef, w1a, w1b, w1c, b1, w2, b2)


def kernel(node_features, edge_index, edge_features, W1, b1, W2, b2):
    src = edge_index[0].astype(jnp.int32)
    tgt = edge_index[1].astype(jnp.int32)
    srcg, tgtg = _sc_gather(node_features, src, tgt)
    w1a = W1[:D_FEAT]
    w1b = W1[D_FEAT:2 * D_FEAT]
    w1c = W1[2 * D_FEAT:]
    return _tc_mlp(srcg, tgtg, edge_features, w1a, w1b, w1c,
                   b1.reshape(1, HIDDEN), W2, b2.reshape(1, OUT))
